# Initial kernel scaffold; baseline (speedup 1.0000x reference)
import jax
import jax.numpy as jnp
from jax.experimental import pallas as pl


def kernel(input, target):
    raise NotImplementedError("write your pallas kernel here")


# speedup vs baseline: 543.7544x; 543.7544x over previous
import functools

import jax
import jax.numpy as jnp
from jax.experimental import pallas as pl
from jax.experimental.pallas import tpu as pltpu

_SCALES = (9, 18, 36)
_WEIGHTS = (0.1, 0.3, 0.6)
_STRIDES = (2, 4, 9)
_DIL = 2
_EPS = 1e-5
_L = 144
_NOUT = (64, 28, 9)
_HB = 16


def _filt_mat(n, k, stride):
    o = jax.lax.broadcasted_iota(jnp.int32, (n, _L), 0)
    i = jax.lax.broadcasted_iota(jnp.int32, (n, _L), 1)
    d = i - o * stride
    m = (d >= 0) & (d < 2 * k) & ((d & 1) == 0)
    return jnp.where(m, jnp.float32(1.0), jnp.float32(0.0))


def _mm(a, b, pattern):
    return jnp.einsum(pattern, a, b,
                      precision=jax.lax.Precision.HIGHEST,
                      preferred_element_type=jnp.float32)


def _stage1_kernel(x_ref, y_ref, o9_ref, o18_ref, o36_ref):
    x = x_ref[0]
    y = y_ref[0]
    chans = (x, y, x * x, y * y, x * y)
    out_refs = (o9_ref, o18_ref, o36_ref)
    for si, (k, st, n) in enumerate(zip(_SCALES, _STRIDES, _NOUT)):
        m = _filt_mat(n, k, st)
        for c, v in enumerate(chans):
            t1 = _mm(m, v, 'od,dhw->ohw')
            t2 = _mm(t1, m, 'ohw,pw->ohp')
            out_refs[si][0, :, c] = t2


def _lncc_partial(ref, k, st, n, w):
    m = _filt_mat(n, k, st)
    f = []
    for c in range(5):
        xc = ref[0][:, c]
        xct = jnp.transpose(xc, (1, 0, 2))
        f.append(_mm(m, xct, 'oh,hdw->odw'))
    i_s, t_s, i2_s, t2_s, it_s = f
    inv_numel = jnp.float32(1.0 / float(k) ** 3)
    numel = jnp.float32(float(k) ** 3)
    i_m = i_s * inv_numel
    t_m = t_s * inv_numel
    cross = it_s - t_m * i_s - i_m * t_s + t_m * i_m * numel
    i_var = i2_s - 2.0 * i_m * i_s + i_m * i_m * numel
    t_var = t2_s - 2.0 * t_m * t_s + t_m * t_m * numel
    lncc = (cross * cross) / (i_var * t_var + jnp.float32(_EPS))
    return jnp.float32(w / float(n) ** 3) * jnp.sum(lncc)


def _stage2_kernel(o9_ref, o18_ref, o36_ref, out_ref):
    c = pl.program_id(1)

    @pl.when(c == 0)
    def _():
        out_ref[0, 0, 0] = _lncc_partial(o36_ref, 36, 9, 9, 0.6)

    q = (_lncc_partial(o9_ref, 9, 2, 64, 0.1)
         + _lncc_partial(o18_ref, 18, 4, 28, 0.3))
    out_ref[0, 0, 0] += q


@jax.jit
def kernel(input, target):
    B = input.shape[0]
    x = input.reshape(B, _L, _L, _L)
    y = target.reshape(B, _L, _L, _L)
    nh = _L // _HB

    out_shapes = [
        jax.ShapeDtypeStruct((B, n, 5, _L, n), jnp.float32) for n in _NOUT
    ]
    in_spec = pl.BlockSpec((1, _L, _HB, _L), lambda b, h: (b, 0, h, 0))
    out_specs = [
        pl.BlockSpec((1, n, 5, _HB, n), lambda b, h: (b, 0, 0, h, 0))
        for n in _NOUT
    ]
    o9, o18, o36 = pl.pallas_call(
        _stage1_kernel,
        grid=(B, nh),
        in_specs=[in_spec, in_spec],
        out_specs=out_specs,
        out_shape=out_shapes,
        compiler_params=pltpu.CompilerParams(
            dimension_semantics=('parallel', 'parallel'),
            vmem_limit_bytes=100 * 1024 * 1024,
        ),
    )(x, y)

    q = pl.pallas_call(
        _stage2_kernel,
        grid=(B, 4),
        in_specs=[
            pl.BlockSpec((1, 16, 5, _L, 64), lambda b, c: (b, c, 0, 0, 0)),
            pl.BlockSpec((1, 7, 5, _L, 28), lambda b, c: (b, c, 0, 0, 0)),
            pl.BlockSpec((1, 9, 5, _L, 9), lambda b, c: (b, 0, 0, 0, 0)),
        ],
        out_specs=pl.BlockSpec((1, 1, 1), lambda b, c: (b, 0, 0),
                               memory_space=pltpu.SMEM),
        out_shape=jax.ShapeDtypeStruct((B, 1, 1), jnp.float32),
        compiler_params=pltpu.CompilerParams(
            dimension_semantics=('parallel', 'arbitrary'),
            vmem_limit_bytes=100 * 1024 * 1024,
        ),
    )(o9, o18, o36)

    return jnp.float32(B) - jnp.sum(q)


# speedup vs baseline: 704.3731x; 1.2954x over previous
import functools

import jax
import jax.numpy as jnp
from jax.experimental import pallas as pl
from jax.experimental.pallas import tpu as pltpu

_SCALES = (9, 18, 36)
_WEIGHTS = (0.1, 0.3, 0.6)
_STRIDES = (2, 4, 9)
_DIL = 2
_EPS = 1e-5
_L = 144
_NOUT = (64, 28, 9)
_HB = 16


def _filt_mat(n, k, stride):
    o = jax.lax.broadcasted_iota(jnp.int32, (n, _L), 0)
    i = jax.lax.broadcasted_iota(jnp.int32, (n, _L), 1)
    d = i - o * stride
    m = (d >= 0) & (d < 2 * k) & ((d & 1) == 0)
    return jnp.where(m, jnp.float32(1.0), jnp.float32(0.0))


def _mm(a, b, pattern, mat_is_lhs=True):
    ein = functools.partial(jnp.einsum, pattern,
                            preferred_element_type=jnp.float32)
    if mat_is_lhs:
        m, v = a, b
    else:
        v, m = a, b
    mb = m.astype(jnp.bfloat16)
    vhi = v.astype(jnp.bfloat16)
    vlo = (v - vhi.astype(jnp.float32)).astype(jnp.bfloat16)
    if mat_is_lhs:
        return ein(mb, vhi) + ein(mb, vlo)
    return ein(vhi, mb) + ein(vlo, mb)


def _stage1_kernel(x_ref, y_ref, o9_ref, o18_ref, o36_ref):
    x = x_ref[0]
    y = y_ref[0]
    chans = (x, y, x * x, y * y, x * y)
    out_refs = (o9_ref, o18_ref, o36_ref)
    for si, (k, st, n) in enumerate(zip(_SCALES, _STRIDES, _NOUT)):
        m = _filt_mat(n, k, st)
        for c, v in enumerate(chans):
            t1 = _mm(m, v, 'od,dhw->ohw')
            t2 = _mm(t1, m, 'ohw,pw->ohp', mat_is_lhs=False)
            out_refs[si][0, :, c] = t2


def _lncc_partial(ref, k, st, n, w):
    m = _filt_mat(n, k, st)
    f = []
    for c in range(5):
        xc = ref[0][:, c]
        f.append(_mm(xc, m, 'dhw,oh->dwo', mat_is_lhs=False))
    i_s, t_s, i2_s, t2_s, it_s = f
    inv_numel = jnp.float32(1.0 / float(k) ** 3)
    numel = jnp.float32(float(k) ** 3)
    i_m = i_s * inv_numel
    t_m = t_s * inv_numel
    cross = it_s - t_m * i_s - i_m * t_s + t_m * i_m * numel
    i_var = i2_s - 2.0 * i_m * i_s + i_m * i_m * numel
    t_var = t2_s - 2.0 * t_m * t_s + t_m * t_m * numel
    lncc = (cross * cross) / (i_var * t_var + jnp.float32(_EPS))
    return jnp.float32(w / float(n) ** 3) * jnp.sum(lncc)


def _stage2_kernel(o9_ref, o18_ref, o36_ref, out_ref):
    c = pl.program_id(1)

    @pl.when(c == 0)
    def _():
        out_ref[0, 0, 0] = _lncc_partial(o36_ref, 36, 9, 9, 0.6)

    q = (_lncc_partial(o9_ref, 9, 2, 64, 0.1)
         + _lncc_partial(o18_ref, 18, 4, 28, 0.3))
    out_ref[0, 0, 0] += q


@jax.jit
def kernel(input, target):
    B = input.shape[0]
    x = input.reshape(B, _L, _L, _L)
    y = target.reshape(B, _L, _L, _L)
    nh = _L // _HB

    out_shapes = [
        jax.ShapeDtypeStruct((B, n, 5, _L, n), jnp.float32) for n in _NOUT
    ]
    in_spec = pl.BlockSpec((1, _L, _HB, _L), lambda b, h: (b, 0, h, 0))
    out_specs = [
        pl.BlockSpec((1, n, 5, _HB, n), lambda b, h: (b, 0, 0, h, 0))
        for n in _NOUT
    ]
    o9, o18, o36 = pl.pallas_call(
        _stage1_kernel,
        grid=(B, nh),
        in_specs=[in_spec, in_spec],
        out_specs=out_specs,
        out_shape=out_shapes,
        compiler_params=pltpu.CompilerParams(
            dimension_semantics=('parallel', 'parallel'),
            vmem_limit_bytes=100 * 1024 * 1024,
        ),
    )(x, y)

    q = pl.pallas_call(
        _stage2_kernel,
        grid=(B, 4),
        in_specs=[
            pl.BlockSpec((1, 16, 5, _L, 64), lambda b, c: (b, c, 0, 0, 0)),
            pl.BlockSpec((1, 7, 5, _L, 28), lambda b, c: (b, c, 0, 0, 0)),
            pl.BlockSpec((1, 9, 5, _L, 9), lambda b, c: (b, 0, 0, 0, 0)),
        ],
        out_specs=pl.BlockSpec((1, 1, 1), lambda b, c: (b, 0, 0),
                               memory_space=pltpu.SMEM),
        out_shape=jax.ShapeDtypeStruct((B, 1, 1), jnp.float32),
        compiler_params=pltpu.CompilerParams(
            dimension_semantics=('parallel', 'arbitrary'),
            vmem_limit_bytes=100 * 1024 * 1024,
        ),
    )(o9, o18, o36)

    return jnp.float32(B) - jnp.sum(q)


# speedup vs baseline: 2293.5961x; 3.2562x over previous
import functools

import jax
import jax.numpy as jnp
from jax.experimental import pallas as pl
from jax.experimental.pallas import tpu as pltpu

_SCALES = (9, 18, 36)
_WEIGHTS = (0.1, 0.3, 0.6)
_STRIDES = (2, 4, 9)
_DIL = 2
_EPS = 1e-5
_L = 144
_NOUT = (64, 28, 9)
_HB = 16


def _filt_mat(n, k, stride):
    o = jax.lax.broadcasted_iota(jnp.int32, (n, _L), 0)
    i = jax.lax.broadcasted_iota(jnp.int32, (n, _L), 1)
    d = i - o * stride
    m = (d >= 0) & (d < 2 * k) & ((d & 1) == 0)
    return jnp.where(m, jnp.float32(1.0), jnp.float32(0.0))


def _mm(a, b, pattern, mat_is_lhs=True):
    ein = functools.partial(jnp.einsum, pattern,
                            preferred_element_type=jnp.float32)
    if mat_is_lhs:
        m, v = a, b
    else:
        v, m = a, b
    mb = m.astype(jnp.bfloat16)
    vhi = v.astype(jnp.bfloat16)
    vlo = (v - vhi.astype(jnp.float32)).astype(jnp.bfloat16)
    if mat_is_lhs:
        return ein(mb, vhi) + ein(mb, vlo)
    return ein(vhi, mb) + ein(vlo, mb)


def _d_filters(v):
    s2 = v[:-2] + v[2:]
    s4 = s2[:-4] + s2[4:]
    s8 = s4[:-8] + s4[8:]
    s9 = s8[:128] + v[16:]
    hb, w = v.shape[1], v.shape[2]
    t9 = s9.reshape(64, 2, hb, w)[:, 0]
    t18 = (s9[:112].reshape(28, 4, hb, w)[:, 0]
           + s9[16:].reshape(28, 4, hb, w)[:, 2])
    t36 = jnp.concatenate(
        [s9[9 * o:9 * o + 1] + s9[9 * o + 18:9 * o + 19]
         + s9[9 * o + 36:9 * o + 37] + s9[9 * o + 54:9 * o + 55]
         for o in range(9)], axis=0)
    return t9, t18, t36


def _stage1_kernel(x_ref, y_ref, o9_ref, o18_ref, o36_ref):
    x = x_ref[0]
    y = y_ref[0]
    chans = (x, y, x * x, y * y, x * y)
    out_refs = (o9_ref, o18_ref, o36_ref)
    for c, v in enumerate(chans):
        t1s = _d_filters(v)
        for si, (k, st, n) in enumerate(zip(_SCALES, _STRIDES, _NOUT)):
            m = _filt_mat(n, k, st)
            t2 = _mm(t1s[si], m, 'ohw,pw->ohp', mat_is_lhs=False)
            out_refs[si][0, :, c] = t2


def _lncc_partial(ref, k, st, n, w):
    m = _filt_mat(n, k, st)
    f = []
    for c in range(5):
        xc = ref[0][:, c]
        f.append(_mm(xc, m, 'dhw,oh->dwo', mat_is_lhs=False))
    i_s, t_s, i2_s, t2_s, it_s = f
    inv_numel = jnp.float32(1.0 / float(k) ** 3)
    numel = jnp.float32(float(k) ** 3)
    i_m = i_s * inv_numel
    t_m = t_s * inv_numel
    cross = it_s - t_m * i_s - i_m * t_s + t_m * i_m * numel
    i_var = i2_s - 2.0 * i_m * i_s + i_m * i_m * numel
    t_var = t2_s - 2.0 * t_m * t_s + t_m * t_m * numel
    lncc = (cross * cross) / (i_var * t_var + jnp.float32(_EPS))
    return jnp.float32(w / float(n) ** 3) * jnp.sum(lncc)


def _stage2_kernel(o9_ref, o18_ref, o36_ref, out_ref):
    c = pl.program_id(1)

    @pl.when(c == 0)
    def _():
        out_ref[0, 0, 0] = _lncc_partial(o36_ref, 36, 9, 9, 0.6)

    q = (_lncc_partial(o9_ref, 9, 2, 64, 0.1)
         + _lncc_partial(o18_ref, 18, 4, 28, 0.3))
    out_ref[0, 0, 0] += q


@jax.jit
def kernel(input, target):
    B = input.shape[0]
    x = input.reshape(B, _L, _L, _L)
    y = target.reshape(B, _L, _L, _L)
    nh = _L // _HB

    out_shapes = [
        jax.ShapeDtypeStruct((B, n, 5, _L, n), jnp.float32) for n in _NOUT
    ]
    in_spec = pl.BlockSpec((1, _L, _HB, _L), lambda b, h: (b, 0, h, 0))
    out_specs = [
        pl.BlockSpec((1, n, 5, _HB, n), lambda b, h: (b, 0, 0, h, 0))
        for n in _NOUT
    ]
    o9, o18, o36 = pl.pallas_call(
        _stage1_kernel,
        grid=(B, nh),
        in_specs=[in_spec, in_spec],
        out_specs=out_specs,
        out_shape=out_shapes,
        compiler_params=pltpu.CompilerParams(
            dimension_semantics=('parallel', 'parallel'),
            vmem_limit_bytes=100 * 1024 * 1024,
        ),
    )(x, y)

    q = pl.pallas_call(
        _stage2_kernel,
        grid=(B, 4),
        in_specs=[
            pl.BlockSpec((1, 16, 5, _L, 64), lambda b, c: (b, c, 0, 0, 0)),
            pl.BlockSpec((1, 7, 5, _L, 28), lambda b, c: (b, c, 0, 0, 0)),
            pl.BlockSpec((1, 9, 5, _L, 9), lambda b, c: (b, 0, 0, 0, 0)),
        ],
        out_specs=pl.BlockSpec((1, 1, 1), lambda b, c: (b, 0, 0),
                               memory_space=pltpu.SMEM),
        out_shape=jax.ShapeDtypeStruct((B, 1, 1), jnp.float32),
        compiler_params=pltpu.CompilerParams(
            dimension_semantics=('parallel', 'arbitrary'),
            vmem_limit_bytes=100 * 1024 * 1024,
        ),
    )(o9, o18, o36)

    return jnp.float32(B) - jnp.sum(q)


# speedup vs baseline: 2857.6811x; 1.2459x over previous
import functools

import jax
import jax.numpy as jnp
from jax.experimental import pallas as pl
from jax.experimental.pallas import tpu as pltpu

_SCALES = (9, 18, 36)
_WEIGHTS = (0.1, 0.3, 0.6)
_STRIDES = (2, 4, 9)
_DIL = 2
_EPS = 1e-5
_L = 144
_NOUT = (64, 28, 9)
_HB = 16


def _filt_mat(n, k, stride):
    o = jax.lax.broadcasted_iota(jnp.int32, (n, _L), 0)
    i = jax.lax.broadcasted_iota(jnp.int32, (n, _L), 1)
    d = i - o * stride
    m = (d >= 0) & (d < 2 * k) & ((d & 1) == 0)
    return jnp.where(m, jnp.float32(1.0), jnp.float32(0.0))


def _mm(a, b, pattern, mat_is_lhs=True, split=True):
    ein = functools.partial(jnp.einsum, pattern,
                            preferred_element_type=jnp.float32)
    if mat_is_lhs:
        m, v = a, b
    else:
        v, m = a, b
    mb = m.astype(jnp.bfloat16)
    if not split:
        return ein(mb, v) if mat_is_lhs else ein(v, mb)
    vhi = v.astype(jnp.bfloat16)
    vlo = (v - vhi.astype(jnp.float32)).astype(jnp.bfloat16)
    if mat_is_lhs:
        return ein(mb, vhi) + ein(mb, vlo)
    return ein(vhi, mb) + ein(vlo, mb)


def _d_filters(v):
    s2 = v[:-2] + v[2:]
    s4 = s2[:-4] + s2[4:]
    s8 = s4[:-8] + s4[8:]
    s9 = s8[:128] + v[16:]
    hb, w = v.shape[1], v.shape[2]
    t9 = s9.reshape(64, 2, hb, w)[:, 0]
    t18 = (s9[:112].reshape(28, 4, hb, w)[:, 0]
           + s9[16:].reshape(28, 4, hb, w)[:, 2])
    t36 = jnp.concatenate(
        [s9[9 * o:9 * o + 1] + s9[9 * o + 18:9 * o + 19]
         + s9[9 * o + 36:9 * o + 37] + s9[9 * o + 54:9 * o + 55]
         for o in range(9)], axis=0)
    return t9, t18, t36


def _stage1_kernel(x_ref, y_ref, o9_ref, o18_ref, o36_ref):
    x = x_ref[0]
    y = y_ref[0]
    chans = (x, y, x * x, y * y, x * y)
    out_refs = (o9_ref, o18_ref, o36_ref)
    for c, v in enumerate(chans):
        t1s = _d_filters(v)
        for si, (k, st, n) in enumerate(zip(_SCALES, _STRIDES, _NOUT)):
            m = _filt_mat(n, k, st)
            t2 = _mm(t1s[si], m, 'ohw,pw->ohp', mat_is_lhs=False)
            out_refs[si][0, :, c] = t2.astype(jnp.bfloat16)


def _lncc_partial(ref, k, st, n, w):
    m = _filt_mat(n, k, st)
    f = []
    for c in range(5):
        xc = ref[0][:, c]
        f.append(_mm(xc, m, 'dhw,oh->dwo', mat_is_lhs=False,
                     split=False))
    i_s, t_s, i2_s, t2_s, it_s = f
    inv_numel = jnp.float32(1.0 / float(k) ** 3)
    numel = jnp.float32(float(k) ** 3)
    i_m = i_s * inv_numel
    t_m = t_s * inv_numel
    cross = it_s - t_m * i_s - i_m * t_s + t_m * i_m * numel
    i_var = i2_s - 2.0 * i_m * i_s + i_m * i_m * numel
    t_var = t2_s - 2.0 * t_m * t_s + t_m * t_m * numel
    lncc = (cross * cross) / (i_var * t_var + jnp.float32(_EPS))
    return jnp.float32(w / float(n) ** 3) * jnp.sum(lncc)


def _stage2_kernel(o9_ref, o18_ref, o36_ref, out_ref):
    c = pl.program_id(1)

    @pl.when(c == 0)
    def _():
        out_ref[0, 0, 0] = _lncc_partial(o36_ref, 36, 9, 9, 0.6)

    q = (_lncc_partial(o9_ref, 9, 2, 64, 0.1)
         + _lncc_partial(o18_ref, 18, 4, 28, 0.3))
    out_ref[0, 0, 0] += q


@jax.jit
def kernel(input, target):
    B = input.shape[0]
    x = input.reshape(B, _L, _L, _L)
    y = target.reshape(B, _L, _L, _L)
    nh = _L // _HB

    out_shapes = [
        jax.ShapeDtypeStruct((B, n, 5, _L, n), jnp.bfloat16) for n in _NOUT
    ]
    in_spec = pl.BlockSpec((1, _L, _HB, _L), lambda b, h: (b, 0, h, 0))
    out_specs = [
        pl.BlockSpec((1, n, 5, _HB, n), lambda b, h: (b, 0, 0, h, 0))
        for n in _NOUT
    ]
    o9, o18, o36 = pl.pallas_call(
        _stage1_kernel,
        grid=(B, nh),
        in_specs=[in_spec, in_spec],
        out_specs=out_specs,
        out_shape=out_shapes,
        compiler_params=pltpu.CompilerParams(
            dimension_semantics=('parallel', 'parallel'),
            vmem_limit_bytes=100 * 1024 * 1024,
        ),
    )(x, y)

    q = pl.pallas_call(
        _stage2_kernel,
        grid=(B, 4),
        in_specs=[
            pl.BlockSpec((1, 16, 5, _L, 64), lambda b, c: (b, c, 0, 0, 0)),
            pl.BlockSpec((1, 7, 5, _L, 28), lambda b, c: (b, c, 0, 0, 0)),
            pl.BlockSpec((1, 9, 5, _L, 9), lambda b, c: (b, 0, 0, 0, 0)),
        ],
        out_specs=pl.BlockSpec((1, 1, 1), lambda b, c: (b, 0, 0),
                               memory_space=pltpu.SMEM),
        out_shape=jax.ShapeDtypeStruct((B, 1, 1), jnp.float32),
        compiler_params=pltpu.CompilerParams(
            dimension_semantics=('parallel', 'arbitrary'),
            vmem_limit_bytes=100 * 1024 * 1024,
        ),
    )(o9, o18, o36)

    return jnp.float32(B) - jnp.sum(q)


# speedup vs baseline: 3130.2865x; 1.0954x over previous
import functools

import jax
import jax.numpy as jnp
from jax.experimental import pallas as pl
from jax.experimental.pallas import tpu as pltpu

_SCALES = (9, 18, 36)
_WEIGHTS = (0.1, 0.3, 0.6)
_STRIDES = (2, 4, 9)
_DIL = 2
_EPS = 1e-5
_L = 144
_NOUT = (64, 28, 9)
_HB = 16


def _filt_mat(n, k, stride):
    o = jax.lax.broadcasted_iota(jnp.int32, (n, _L), 0)
    i = jax.lax.broadcasted_iota(jnp.int32, (n, _L), 1)
    d = i - o * stride
    m = (d >= 0) & (d < 2 * k) & ((d & 1) == 0)
    return jnp.where(m, jnp.float32(1.0), jnp.float32(0.0))


def _mm(a, b, pattern, mat_is_lhs=True, split=True):
    ein = functools.partial(jnp.einsum, pattern,
                            preferred_element_type=jnp.float32)
    if mat_is_lhs:
        m, v = a, b
    else:
        v, m = a, b
    mb = m.astype(jnp.bfloat16)
    if not split:
        return ein(mb, v) if mat_is_lhs else ein(v, mb)
    vhi = v.astype(jnp.bfloat16)
    vlo = (v - vhi.astype(jnp.float32)).astype(jnp.bfloat16)
    if mat_is_lhs:
        return ein(mb, vhi) + ein(mb, vlo)
    return ein(vhi, mb) + ein(vlo, mb)


def _d_filters(v):
    s2 = v[:-2] + v[2:]
    s4 = s2[:-4] + s2[4:]
    s8 = s4[:-8] + s4[8:]
    s9 = s8[:128] + v[16:]
    hb, w = v.shape[1], v.shape[2]
    t9 = s9.reshape(64, 2, hb, w)[:, 0]
    t18 = (s9[:112].reshape(28, 4, hb, w)[:, 0]
           + s9[16:].reshape(28, 4, hb, w)[:, 2])
    t36 = jnp.concatenate(
        [s9[9 * o:9 * o + 1] + s9[9 * o + 18:9 * o + 19]
         + s9[9 * o + 36:9 * o + 37] + s9[9 * o + 54:9 * o + 55]
         for o in range(9)], axis=0)
    return t9, t18, t36


def _stage1_kernel(x_ref, y_ref, o9_ref, o18_ref, o36_ref):
    x = x_ref[0]
    y = y_ref[0]
    chans = (x, y, x * x, y * y, x * y)
    out_refs = (o9_ref, o18_ref, o36_ref)
    for c, v in enumerate(chans):
        t1s = _d_filters(v)
        for si, (k, st, n) in enumerate(zip(_SCALES, _STRIDES, _NOUT)):
            m = _filt_mat(n, k, st)
            t2 = _mm(t1s[si].astype(jnp.bfloat16), m, 'ohw,pw->ohp',
                     mat_is_lhs=False, split=False)
            out_refs[si][0, :, c] = t2.astype(jnp.bfloat16)


def _lncc_partial(ref, k, st, n, w):
    m = _filt_mat(n, k, st)
    f = []
    for c in range(5):
        xc = ref[0][:, c]
        f.append(_mm(xc, m, 'dhw,oh->dwo', mat_is_lhs=False,
                     split=False))
    i_s, t_s, i2_s, t2_s, it_s = f
    inv_numel = jnp.float32(1.0 / float(k) ** 3)
    numel = jnp.float32(float(k) ** 3)
    i_m = i_s * inv_numel
    t_m = t_s * inv_numel
    cross = it_s - t_m * i_s - i_m * t_s + t_m * i_m * numel
    i_var = i2_s - 2.0 * i_m * i_s + i_m * i_m * numel
    t_var = t2_s - 2.0 * t_m * t_s + t_m * t_m * numel
    lncc = (cross * cross) / (i_var * t_var + jnp.float32(_EPS))
    return jnp.float32(w / float(n) ** 3) * jnp.sum(lncc)


def _stage2_kernel(o9_ref, o18_ref, o36_ref, out_ref):
    c = pl.program_id(1)

    @pl.when(c == 0)
    def _():
        out_ref[0, 0, 0] = _lncc_partial(o36_ref, 36, 9, 9, 0.6)

    q = (_lncc_partial(o9_ref, 9, 2, 64, 0.1)
         + _lncc_partial(o18_ref, 18, 4, 28, 0.3))
    out_ref[0, 0, 0] += q


@jax.jit
def kernel(input, target):
    B = input.shape[0]
    x = input.reshape(B, _L, _L, _L)
    y = target.reshape(B, _L, _L, _L)
    nh = _L // _HB

    out_shapes = [
        jax.ShapeDtypeStruct((B, n, 5, _L, n), jnp.bfloat16) for n in _NOUT
    ]
    in_spec = pl.BlockSpec((1, _L, _HB, _L), lambda b, h: (b, 0, h, 0))
    out_specs = [
        pl.BlockSpec((1, n, 5, _HB, n), lambda b, h: (b, 0, 0, h, 0))
        for n in _NOUT
    ]
    o9, o18, o36 = pl.pallas_call(
        _stage1_kernel,
        grid=(B, nh),
        in_specs=[in_spec, in_spec],
        out_specs=out_specs,
        out_shape=out_shapes,
        compiler_params=pltpu.CompilerParams(
            dimension_semantics=('parallel', 'parallel'),
            vmem_limit_bytes=100 * 1024 * 1024,
        ),
    )(x, y)

    q = pl.pallas_call(
        _stage2_kernel,
        grid=(B, 4),
        in_specs=[
            pl.BlockSpec((1, 16, 5, _L, 64), lambda b, c: (b, c, 0, 0, 0)),
            pl.BlockSpec((1, 7, 5, _L, 28), lambda b, c: (b, c, 0, 0, 0)),
            pl.BlockSpec((1, 9, 5, _L, 9), lambda b, c: (b, 0, 0, 0, 0)),
        ],
        out_specs=pl.BlockSpec((1, 1, 1), lambda b, c: (b, 0, 0),
                               memory_space=pltpu.SMEM),
        out_shape=jax.ShapeDtypeStruct((B, 1, 1), jnp.float32),
        compiler_params=pltpu.CompilerParams(
            dimension_semantics=('parallel', 'arbitrary'),
            vmem_limit_bytes=100 * 1024 * 1024,
        ),
    )(o9, o18, o36)

    return jnp.float32(B) - jnp.sum(q)


# speedup vs baseline: 3239.8147x; 1.0350x over previous
import functools

import jax
import jax.numpy as jnp
from jax.experimental import pallas as pl
from jax.experimental.pallas import tpu as pltpu

_SCALES = (9, 18, 36)
_WEIGHTS = (0.1, 0.3, 0.6)
_STRIDES = (2, 4, 9)
_DIL = 2
_EPS = 1e-5
_L = 144
_NOUT = (64, 28, 9)
_HB = 24


def _filt_mat(n, k, stride):
    o = jax.lax.broadcasted_iota(jnp.int32, (n, _L), 0)
    i = jax.lax.broadcasted_iota(jnp.int32, (n, _L), 1)
    d = i - o * stride
    m = (d >= 0) & (d < 2 * k) & ((d & 1) == 0)
    return jnp.where(m, jnp.float32(1.0), jnp.float32(0.0))


def _mm(a, b, pattern, mat_is_lhs=True, split=True):
    ein = functools.partial(jnp.einsum, pattern,
                            preferred_element_type=jnp.float32)
    if mat_is_lhs:
        m, v = a, b
    else:
        v, m = a, b
    mb = m.astype(jnp.bfloat16)
    if not split:
        return ein(mb, v) if mat_is_lhs else ein(v, mb)
    vhi = v.astype(jnp.bfloat16)
    vlo = (v - vhi.astype(jnp.float32)).astype(jnp.bfloat16)
    if mat_is_lhs:
        return ein(mb, vhi) + ein(mb, vlo)
    return ein(vhi, mb) + ein(vlo, mb)


def _d_filters(v):
    s2 = v[:-2] + v[2:]
    s4 = s2[:-4] + s2[4:]
    s8 = s4[:-8] + s4[8:]
    s9 = s8[:128] + v[16:]
    hb, w = v.shape[1], v.shape[2]
    t9 = s9.reshape(64, 2, hb, w)[:, 0]
    t18 = (s9[:112].reshape(28, 4, hb, w)[:, 0]
           + s9[16:].reshape(28, 4, hb, w)[:, 2])
    t36 = jnp.concatenate(
        [s9[9 * o:9 * o + 1] + s9[9 * o + 18:9 * o + 19]
         + s9[9 * o + 36:9 * o + 37] + s9[9 * o + 54:9 * o + 55]
         for o in range(9)], axis=0)
    return t9, t18, t36


def _stage1_kernel(x_ref, y_ref, o9_ref, o18_ref, o36_ref):
    x = x_ref[0]
    y = y_ref[0]
    chans = (x, y, x * x, y * y, x * y)
    out_refs = (o9_ref, o18_ref, o36_ref)
    for c, v in enumerate(chans):
        t1s = _d_filters(v)
        for si, (k, st, n) in enumerate(zip(_SCALES, _STRIDES, _NOUT)):
            m = _filt_mat(n, k, st)
            t2 = _mm(t1s[si].astype(jnp.bfloat16), m, 'ohw,pw->ohp',
                     mat_is_lhs=False, split=False)
            out_refs[si][0, :, c] = t2.astype(jnp.bfloat16)


def _lncc_partial(ref, k, st, n, w):
    m = _filt_mat(n, k, st)
    f = []
    for c in range(5):
        xc = ref[0][:, c]
        f.append(_mm(xc, m, 'dhw,oh->dwo', mat_is_lhs=False,
                     split=False))
    i_s, t_s, i2_s, t2_s, it_s = f
    inv_numel = jnp.float32(1.0 / float(k) ** 3)
    numel = jnp.float32(float(k) ** 3)
    i_m = i_s * inv_numel
    t_m = t_s * inv_numel
    cross = it_s - t_m * i_s - i_m * t_s + t_m * i_m * numel
    i_var = i2_s - 2.0 * i_m * i_s + i_m * i_m * numel
    t_var = t2_s - 2.0 * t_m * t_s + t_m * t_m * numel
    lncc = (cross * cross) / (i_var * t_var + jnp.float32(_EPS))
    return jnp.float32(w / float(n) ** 3) * jnp.sum(lncc)


def _stage2_kernel(o9_ref, o18_ref, o36_ref, out_ref):
    c = pl.program_id(1)

    @pl.when(c == 0)
    def _():
        out_ref[0, 0, 0] = _lncc_partial(o36_ref, 36, 9, 9, 0.6)

    q = (_lncc_partial(o9_ref, 9, 2, 64, 0.1)
         + _lncc_partial(o18_ref, 18, 4, 28, 0.3))
    out_ref[0, 0, 0] += q


@jax.jit
def kernel(input, target):
    B = input.shape[0]
    x = input.reshape(B, _L, _L, _L)
    y = target.reshape(B, _L, _L, _L)
    nh = _L // _HB

    out_shapes = [
        jax.ShapeDtypeStruct((B, n, 5, _L, n), jnp.bfloat16) for n in _NOUT
    ]
    in_spec = pl.BlockSpec((1, _L, _HB, _L), lambda b, h: (b, 0, h, 0))
    out_specs = [
        pl.BlockSpec((1, n, 5, _HB, n), lambda b, h: (b, 0, 0, h, 0))
        for n in _NOUT
    ]
    o9, o18, o36 = pl.pallas_call(
        _stage1_kernel,
        grid=(B, nh),
        in_specs=[in_spec, in_spec],
        out_specs=out_specs,
        out_shape=out_shapes,
        compiler_params=pltpu.CompilerParams(
            dimension_semantics=('parallel', 'parallel'),
            vmem_limit_bytes=100 * 1024 * 1024,
        ),
    )(x, y)

    q = pl.pallas_call(
        _stage2_kernel,
        grid=(B, 4),
        in_specs=[
            pl.BlockSpec((1, 16, 5, _L, 64), lambda b, c: (b, c, 0, 0, 0)),
            pl.BlockSpec((1, 7, 5, _L, 28), lambda b, c: (b, c, 0, 0, 0)),
            pl.BlockSpec((1, 9, 5, _L, 9), lambda b, c: (b, 0, 0, 0, 0)),
        ],
        out_specs=pl.BlockSpec((1, 1, 1), lambda b, c: (b, 0, 0),
                               memory_space=pltpu.SMEM),
        out_shape=jax.ShapeDtypeStruct((B, 1, 1), jnp.float32),
        compiler_params=pltpu.CompilerParams(
            dimension_semantics=('parallel', 'arbitrary'),
            vmem_limit_bytes=100 * 1024 * 1024,
        ),
    )(o9, o18, o36)

    return jnp.float32(B) - jnp.sum(q)
